# Initial kernel scaffold; baseline (speedup 1.0000x reference)
import jax
import jax.numpy as jnp
from jax.experimental import pallas as pl


def kernel(x, edge_index, W, b):
    raise NotImplementedError("write your pallas kernel here")


# speedup vs baseline: 34.8504x; 34.8504x over previous
import functools

import jax
import jax.numpy as jnp
from jax import lax
from jax.experimental import pallas as pl
from jax.experimental.pallas import tpu as pltpu
from jax.experimental.pallas import tpu_sc as plsc

N_NODES = 10000
NP = 10240
C = 16
NSC = 2
NTILE = 16
NW = NSC * NTILE
CHUNK = 128
ROWS_PER_TILE = NP // NTILE


def _sc_mesh():
    return plsc.VectorSubcoreMesh(core_axis_name="c", subcore_axis_name="s")


def _sc_degree(nchunks):
    @functools.partial(
        pl.kernel,
        out_type=jax.ShapeDtypeStruct((NSC, NP, C), jnp.float32),
        mesh=_sc_mesh(),
        compiler_params=pltpu.CompilerParams(use_tc_tiling_on_sc=False),
        scratch_types=[
            pltpu.VMEM((nchunks, CHUNK), jnp.int32),
            pltpu.VMEM((CHUNK, C), jnp.float32),
            pltpu.VMEM_SHARED((NP, C), jnp.float32),
        ],
    )
    def deg_kernel(dst_hbm, zeros_hbm, out_hbm, didx_v, ones_v, acc):
        cid = lax.axis_index("c")
        sid = lax.axis_index("s")
        wid = cid * NTILE + sid
        row0 = sid * ROWS_PER_TILE
        pltpu.sync_copy(zeros_hbm.at[pl.ds(row0, ROWS_PER_TILE)],
                        acc.at[pl.ds(row0, ROWS_PER_TILE)])
        pltpu.sync_copy(dst_hbm.at[wid], didx_v)

        def fill(i, carry):
            ones_v[i, :] = jnp.ones((C,), jnp.float32)
            return carry
        lax.fori_loop(0, CHUNK, fill, 0)
        plsc.subcore_barrier()

        def body(j, carry):
            pltpu.sync_copy(ones_v, acc.at[didx_v.at[j]], add=True)
            return carry
        lax.fori_loop(0, nchunks, body, 0)
        plsc.subcore_barrier()
        pltpu.sync_copy(acc.at[pl.ds(row0, ROWS_PER_TILE)],
                        out_hbm.at[cid, pl.ds(row0, ROWS_PER_TILE)])

    return deg_kernel


def _sc_propagate(nchunks):
    @functools.partial(
        pl.kernel,
        out_type=jax.ShapeDtypeStruct((NSC, NP, C), jnp.float32),
        mesh=_sc_mesh(),
        compiler_params=pltpu.CompilerParams(use_tc_tiling_on_sc=False),
        scratch_types=[
            pltpu.VMEM((nchunks, CHUNK), jnp.int32),
            pltpu.VMEM((nchunks, CHUNK), jnp.int32),
            pltpu.VMEM((CHUNK, C), jnp.float32),
            pltpu.VMEM_SHARED((NP, C), jnp.float32),
            pltpu.SemaphoreType.DMA,
        ],
    )
    def prop_kernel(g_hbm, src_hbm, dst_hbm, zeros_hbm, out_hbm,
                    sidx_v, didx_v, rows_v, acc, sem):
        cid = lax.axis_index("c")
        sid = lax.axis_index("s")
        wid = cid * NTILE + sid
        row0 = sid * ROWS_PER_TILE
        pltpu.sync_copy(zeros_hbm.at[pl.ds(row0, ROWS_PER_TILE)],
                        acc.at[pl.ds(row0, ROWS_PER_TILE)])
        pltpu.sync_copy(src_hbm.at[wid], sidx_v)
        pltpu.sync_copy(dst_hbm.at[wid], didx_v)
        plsc.subcore_barrier()

        def body(j, carry):
            pltpu.async_copy(g_hbm.at[sidx_v.at[j]], rows_v, sem).wait()
            pltpu.sync_copy(rows_v, acc.at[didx_v.at[j]], add=True)
            return carry
        lax.fori_loop(0, nchunks, body, 0)
        plsc.subcore_barrier()
        pltpu.sync_copy(acc.at[pl.ds(row0, ROWS_PER_TILE)],
                        out_hbm.at[cid, pl.ds(row0, ROWS_PER_TILE)])

    return prop_kernel


_BR = 1024


def _deg_col(dp_ref):
    return dp_ref[0, :, 0:1] + dp_ref[1, :, 0:1] + 1.0


def _tc_project(x_pad, w, degp):
    def body(x_ref, w_ref, dp_ref, g_ref):
        y = lax.dot_general(x_ref[...], w_ref[...], (((1,), (1,)), ((), ())),
                            preferred_element_type=jnp.float32)
        g_ref[...] = y * lax.rsqrt(_deg_col(dp_ref))

    return pl.pallas_call(
        body,
        grid=(NP // _BR,),
        in_specs=[
            pl.BlockSpec((_BR, 128), lambda i: (i, 0)),
            pl.BlockSpec((C, 128), lambda i: (0, 0)),
            pl.BlockSpec((NSC, _BR, C), lambda i: (0, i, 0)),
        ],
        out_specs=pl.BlockSpec((_BR, C), lambda i: (i, 0)),
        out_shape=jax.ShapeDtypeStruct((NP, C), jnp.float32),
    )(x_pad, w, degp)


def _tc_combine(qp, g, degp):
    def body(q_ref, g_ref, dp_ref, t_ref):
        t_ref[...] = (q_ref[0] + q_ref[1] + g_ref[...]) / _deg_col(dp_ref)

    return pl.pallas_call(
        body,
        grid=(NP // _BR,),
        in_specs=[
            pl.BlockSpec((NSC, _BR, C), lambda i: (0, i, 0)),
            pl.BlockSpec((_BR, C), lambda i: (i, 0)),
            pl.BlockSpec((NSC, _BR, C), lambda i: (0, i, 0)),
        ],
        out_specs=pl.BlockSpec((_BR, C), lambda i: (i, 0)),
        out_shape=jax.ShapeDtypeStruct((NP, C), jnp.float32),
    )(qp, g, degp)


def _tc_final(rp, t, degp, b2):
    def body(r_ref, t_ref, dp_ref, b_ref, o_ref):
        s = r_ref[0] + r_ref[1] + t_ref[...]
        o_ref[...] = s * lax.rsqrt(_deg_col(dp_ref)) + b_ref[...]

    return pl.pallas_call(
        body,
        grid=(NP // _BR,),
        in_specs=[
            pl.BlockSpec((NSC, _BR, C), lambda i: (0, i, 0)),
            pl.BlockSpec((_BR, C), lambda i: (i, 0)),
            pl.BlockSpec((NSC, _BR, C), lambda i: (0, i, 0)),
            pl.BlockSpec((1, C), lambda i: (0, 0)),
        ],
        out_specs=pl.BlockSpec((_BR, C), lambda i: (i, 0)),
        out_shape=jax.ShapeDtypeStruct((NP, C), jnp.float32),
    )(rp, t, degp, b2)


def kernel(x, edge_index, W, b):
    E = edge_index.shape[1]
    per_w = -(-E // (NW * CHUNK)) * CHUNK
    nchunks = per_w // CHUNK
    e_pad = NW * per_w

    ei = edge_index.astype(jnp.int32)
    pad = jnp.full((2, e_pad - E), N_NODES, jnp.int32)
    ei = jnp.concatenate([ei, pad], axis=1)
    src = ei[0].reshape(NW, nchunks, CHUNK)
    dst = ei[1].reshape(NW, nchunks, CHUNK)
    zeros = jnp.zeros((NP, C), jnp.float32)
    x_pad = jnp.pad(x, ((0, NP - x.shape[0]), (0, 0)))

    degp = _sc_degree(nchunks)(dst, zeros)
    g = _tc_project(x_pad, W, degp)
    qp = _sc_propagate(nchunks)(g, src, dst, zeros)
    t = _tc_combine(qp, g, degp)
    rp = _sc_propagate(nchunks)(t, src, dst, zeros)
    out = _tc_final(rp, t, degp, b.reshape(1, C))
    return out[:N_NODES]


# speedup vs baseline: 37.7279x; 1.0826x over previous
import functools

import jax
import jax.numpy as jnp
from jax import lax
from jax.experimental import pallas as pl
from jax.experimental.pallas import tpu as pltpu
from jax.experimental.pallas import tpu_sc as plsc

N_NODES = 10000
NP = 10240
C = 16
NSC = 2
NTILE = 16
NW = NSC * NTILE
CHUNK = 128
U = 4
ROWS_PER_TILE = NP // NTILE


def _zero_fill(buf, n):
    def fill(i, carry):
        buf[i, :] = jnp.zeros((C,), jnp.float32)
        return carry
    lax.fori_loop(0, n, fill, 0)


def _init_acc_zero(acc, zbuf, row0):
    _zero_fill(zbuf, CHUNK)
    for r in range(ROWS_PER_TILE // CHUNK):
        pltpu.sync_copy(zbuf, acc.at[pl.ds(row0 + r * CHUNK, CHUNK)])


def _sc_mesh():
    return plsc.VectorSubcoreMesh(core_axis_name="c", subcore_axis_name="s")


def _sc_degree(nchunks):
    @functools.partial(
        pl.kernel,
        out_type=jax.ShapeDtypeStruct((NSC, NP, C), jnp.float32),
        mesh=_sc_mesh(),
        compiler_params=pltpu.CompilerParams(use_tc_tiling_on_sc=False),
        scratch_types=[
            pltpu.VMEM((nchunks, CHUNK), jnp.int32),
            pltpu.VMEM((CHUNK, C), jnp.float32),
            pltpu.VMEM((CHUNK, C), jnp.float32),
            pltpu.VMEM_SHARED((NP, C), jnp.float32),
            pltpu.SemaphoreType.DMA,
        ],
    )
    def deg_kernel(dst_hbm, out_hbm, didx_v, ones_v, zbuf, acc, sem):
        cid = lax.axis_index("c")
        sid = lax.axis_index("s")
        wid = cid * NTILE + sid
        row0 = sid * ROWS_PER_TILE
        _init_acc_zero(acc, zbuf, row0)
        pltpu.sync_copy(dst_hbm.at[wid], didx_v)

        def fill(i, carry):
            ones_v[i, :] = jnp.ones((C,), jnp.float32)
            return carry
        lax.fori_loop(0, CHUNK, fill, 0)
        plsc.subcore_barrier()

        def body(blk, carry):
            descs = [
                pltpu.async_copy(ones_v, acc.at[didx_v.at[blk * U + u]],
                                 sem, add=True)
                for u in range(U)
            ]
            for d in descs:
                d.wait()
            return carry
        lax.fori_loop(0, nchunks // U, body, 0)
        plsc.subcore_barrier()
        pltpu.sync_copy(acc.at[pl.ds(row0, ROWS_PER_TILE)],
                        out_hbm.at[cid, pl.ds(row0, ROWS_PER_TILE)])

    return deg_kernel


def _sc_propagate(nchunks):
    @functools.partial(
        pl.kernel,
        out_type=jax.ShapeDtypeStruct((NSC, NP, C), jnp.float32),
        mesh=_sc_mesh(),
        compiler_params=pltpu.CompilerParams(use_tc_tiling_on_sc=False),
        scratch_types=[
            pltpu.VMEM((nchunks, CHUNK), jnp.int32),
            pltpu.VMEM((nchunks, CHUNK), jnp.int32),
            [pltpu.VMEM((CHUNK, C), jnp.float32) for _ in range(U)],
            pltpu.VMEM_SHARED((NP, C), jnp.float32),
            [pltpu.SemaphoreType.DMA for _ in range(U)],
        ],
    )
    def prop_kernel(g_hbm, src_hbm, dst_hbm, out_hbm,
                    sidx_v, didx_v, rows_v, acc, sems):
        cid = lax.axis_index("c")
        sid = lax.axis_index("s")
        wid = cid * NTILE + sid
        row0 = sid * ROWS_PER_TILE
        _init_acc_zero(acc, rows_v[0], row0)
        pltpu.sync_copy(src_hbm.at[wid], sidx_v)
        pltpu.sync_copy(dst_hbm.at[wid], didx_v)
        plsc.subcore_barrier()

        def body(blk, carry):
            descs = [
                pltpu.async_copy(g_hbm.at[sidx_v.at[blk * U + u]],
                                 rows_v[u], sems[u])
                for u in range(U)
            ]
            for u in range(U):
                descs[u].wait()
                pltpu.sync_copy(rows_v[u], acc.at[didx_v.at[blk * U + u]],
                                add=True)
            return carry
        lax.fori_loop(0, nchunks // U, body, 0)
        plsc.subcore_barrier()
        pltpu.sync_copy(acc.at[pl.ds(row0, ROWS_PER_TILE)],
                        out_hbm.at[cid, pl.ds(row0, ROWS_PER_TILE)])

    return prop_kernel


_BR = 1024


def _deg_col(dp_ref):
    return dp_ref[0, :, 0:1] + dp_ref[1, :, 0:1] + 1.0


def _tc_project(x_pad, w, degp):
    def body(x_ref, w_ref, dp_ref, g_ref):
        y = lax.dot_general(x_ref[...], w_ref[...], (((1,), (1,)), ((), ())),
                            preferred_element_type=jnp.float32)
        g_ref[...] = y * lax.rsqrt(_deg_col(dp_ref))

    return pl.pallas_call(
        body,
        grid=(NP // _BR,),
        in_specs=[
            pl.BlockSpec((_BR, 128), lambda i: (i, 0)),
            pl.BlockSpec((C, 128), lambda i: (0, 0)),
            pl.BlockSpec((NSC, _BR, C), lambda i: (0, i, 0)),
        ],
        out_specs=pl.BlockSpec((_BR, C), lambda i: (i, 0)),
        out_shape=jax.ShapeDtypeStruct((NP, C), jnp.float32),
    )(x_pad, w, degp)


def _tc_combine(qp, g, degp):
    def body(q_ref, g_ref, dp_ref, t_ref):
        t_ref[...] = (q_ref[0] + q_ref[1] + g_ref[...]) / _deg_col(dp_ref)

    return pl.pallas_call(
        body,
        grid=(NP // _BR,),
        in_specs=[
            pl.BlockSpec((NSC, _BR, C), lambda i: (0, i, 0)),
            pl.BlockSpec((_BR, C), lambda i: (i, 0)),
            pl.BlockSpec((NSC, _BR, C), lambda i: (0, i, 0)),
        ],
        out_specs=pl.BlockSpec((_BR, C), lambda i: (i, 0)),
        out_shape=jax.ShapeDtypeStruct((NP, C), jnp.float32),
    )(qp, g, degp)


def _tc_final(rp, t, degp, b2):
    def body(r_ref, t_ref, dp_ref, b_ref, o_ref):
        s = r_ref[0] + r_ref[1] + t_ref[...]
        o_ref[...] = s * lax.rsqrt(_deg_col(dp_ref)) + b_ref[...]

    return pl.pallas_call(
        body,
        grid=(NP // _BR,),
        in_specs=[
            pl.BlockSpec((NSC, _BR, C), lambda i: (0, i, 0)),
            pl.BlockSpec((_BR, C), lambda i: (i, 0)),
            pl.BlockSpec((NSC, _BR, C), lambda i: (0, i, 0)),
            pl.BlockSpec((1, C), lambda i: (0, 0)),
        ],
        out_specs=pl.BlockSpec((_BR, C), lambda i: (i, 0)),
        out_shape=jax.ShapeDtypeStruct((NP, C), jnp.float32),
    )(rp, t, degp, b2)


def kernel(x, edge_index, W, b):
    E = edge_index.shape[1]
    per_w = -(-E // (NW * CHUNK * U)) * CHUNK * U
    nchunks = per_w // CHUNK
    e_pad = NW * per_w

    ei = edge_index.astype(jnp.int32)
    pad = jnp.full((2, e_pad - E), N_NODES, jnp.int32)
    ei = jnp.concatenate([ei, pad], axis=1)
    src = ei[0].reshape(NW, nchunks, CHUNK)
    dst = ei[1].reshape(NW, nchunks, CHUNK)
    x_pad = jnp.pad(x, ((0, NP - x.shape[0]), (0, 0)))

    degp = _sc_degree(nchunks)(dst)
    g = _tc_project(x_pad, W, degp)
    qp = _sc_propagate(nchunks)(g, src, dst)
    t = _tc_combine(qp, g, degp)
    rp = _sc_propagate(nchunks)(t, src, dst)
    out = _tc_final(rp, t, degp, b.reshape(1, C))
    return out[:N_NODES]


# speedup vs baseline: 40.7914x; 1.0812x over previous
import functools

import jax
import jax.numpy as jnp
from jax import lax
from jax.experimental import pallas as pl
from jax.experimental.pallas import tpu as pltpu
from jax.experimental.pallas import tpu_sc as plsc

N_NODES = 10000
NP = 10240
C = 16
NSC = 2
NTILE = 16
NW = NSC * NTILE
CHUNK = 128
U = 4
ROWS_PER_TILE = NP // NTILE


def _zero_fill(buf, n):
    def fill(i, carry):
        buf[i, :] = jnp.zeros((C,), jnp.float32)
        return carry
    lax.fori_loop(0, n, fill, 0)


def _init_acc_zero(acc, zbuf, row0):
    _zero_fill(zbuf, CHUNK)
    for r in range(ROWS_PER_TILE // CHUNK):
        pltpu.sync_copy(zbuf, acc.at[pl.ds(row0 + r * CHUNK, CHUNK)])


def _sc_mesh():
    return plsc.VectorSubcoreMesh(core_axis_name="c", subcore_axis_name="s")


def _sc_degree(nblocks, burst):
    @functools.partial(
        pl.kernel,
        out_type=jax.ShapeDtypeStruct((NSC, NP, C), jnp.float32),
        mesh=_sc_mesh(),
        compiler_params=pltpu.CompilerParams(use_tc_tiling_on_sc=False),
        scratch_types=[
            pltpu.VMEM((nblocks, burst * CHUNK), jnp.int32),
            pltpu.VMEM((burst * CHUNK, C), jnp.float32),
            pltpu.VMEM((CHUNK, C), jnp.float32),
            pltpu.VMEM_SHARED((NP, C), jnp.float32),
            pltpu.SemaphoreType.DMA,
        ],
    )
    def deg_kernel(dst_hbm, ones_hbm, out_hbm, didx_v, ones_v, zbuf, acc, sem):
        cid = lax.axis_index("c")
        sid = lax.axis_index("s")
        wid = cid * NTILE + sid
        row0 = sid * ROWS_PER_TILE
        _init_acc_zero(acc, zbuf, row0)
        pltpu.sync_copy(dst_hbm.at[wid], didx_v)
        pltpu.sync_copy(ones_hbm, ones_v)
        plsc.subcore_barrier()

        descs = [
            pltpu.async_copy(ones_v, acc.at[didx_v.at[blk]], sem, add=True)
            for blk in range(nblocks)
        ]
        for d in descs:
            d.wait()
        plsc.subcore_barrier()
        pltpu.sync_copy(acc.at[pl.ds(row0, ROWS_PER_TILE)],
                        out_hbm.at[cid, pl.ds(row0, ROWS_PER_TILE)])

    return deg_kernel


def _sc_propagate(nblocks, burst):
    @functools.partial(
        pl.kernel,
        out_type=jax.ShapeDtypeStruct((NSC, NP, C), jnp.float32),
        mesh=_sc_mesh(),
        compiler_params=pltpu.CompilerParams(use_tc_tiling_on_sc=False),
        scratch_types=[
            pltpu.VMEM((nblocks, burst * CHUNK), jnp.int32),
            pltpu.VMEM((nblocks, burst * CHUNK), jnp.int32),
            [pltpu.VMEM((burst * CHUNK, C), jnp.float32) for _ in range(2)],
            pltpu.VMEM_SHARED((NP, C), jnp.float32),
            [pltpu.SemaphoreType.DMA for _ in range(4)],
        ],
    )
    def prop_kernel(g_hbm, src_hbm, dst_hbm, out_hbm,
                    sidx_v, didx_v, rows_v, acc, sems):
        cid = lax.axis_index("c")
        sid = lax.axis_index("s")
        wid = cid * NTILE + sid
        row0 = sid * ROWS_PER_TILE
        _init_acc_zero(acc, rows_v[0].at[pl.ds(0, CHUNK)], row0)
        pltpu.sync_copy(src_hbm.at[wid], sidx_v)
        pltpu.sync_copy(dst_hbm.at[wid], didx_v)
        plsc.subcore_barrier()

        def gather(blk, u):
            return pltpu.async_copy(g_hbm.at[sidx_v.at[blk]], rows_v[u],
                                    sems[u])

        def scatter(blk, u):
            return pltpu.async_copy(rows_v[u], acc.at[didx_v.at[blk]],
                                    sems[2 + u], add=True)

        g_descs = {0: gather(0, 0)}
        s_descs = {}
        for blk in range(nblocks):
            u = blk % 2
            g_descs[blk].wait()
            s_descs[blk] = scatter(blk, u)
            if blk + 1 < nblocks:
                if blk >= 1:
                    s_descs[blk - 1].wait()
                g_descs[blk + 1] = gather(blk + 1, 1 - u)
        if nblocks >= 2:
            s_descs[nblocks - 2].wait()
        s_descs[nblocks - 1].wait()
        plsc.subcore_barrier()
        pltpu.sync_copy(acc.at[pl.ds(row0, ROWS_PER_TILE)],
                        out_hbm.at[cid, pl.ds(row0, ROWS_PER_TILE)])

    return prop_kernel


_BR = 1024


def _deg_col(dp_ref):
    return dp_ref[0, :, 0:1] + dp_ref[1, :, 0:1] + 1.0


def _tc_project(x_pad, w, degp):
    def body(x_ref, w_ref, dp_ref, g_ref):
        y = lax.dot_general(x_ref[...], w_ref[...], (((1,), (1,)), ((), ())),
                            preferred_element_type=jnp.float32)
        g_ref[...] = y * lax.rsqrt(_deg_col(dp_ref))

    return pl.pallas_call(
        body,
        grid=(NP // _BR,),
        in_specs=[
            pl.BlockSpec((_BR, 128), lambda i: (i, 0)),
            pl.BlockSpec((C, 128), lambda i: (0, 0)),
            pl.BlockSpec((NSC, _BR, C), lambda i: (0, i, 0)),
        ],
        out_specs=pl.BlockSpec((_BR, C), lambda i: (i, 0)),
        out_shape=jax.ShapeDtypeStruct((NP, C), jnp.float32),
    )(x_pad, w, degp)


def _tc_combine(qp, g, degp):
    def body(q_ref, g_ref, dp_ref, t_ref):
        t_ref[...] = (q_ref[0] + q_ref[1] + g_ref[...]) / _deg_col(dp_ref)

    return pl.pallas_call(
        body,
        grid=(NP // _BR,),
        in_specs=[
            pl.BlockSpec((NSC, _BR, C), lambda i: (0, i, 0)),
            pl.BlockSpec((_BR, C), lambda i: (i, 0)),
            pl.BlockSpec((NSC, _BR, C), lambda i: (0, i, 0)),
        ],
        out_specs=pl.BlockSpec((_BR, C), lambda i: (i, 0)),
        out_shape=jax.ShapeDtypeStruct((NP, C), jnp.float32),
    )(qp, g, degp)


def _tc_final(rp, t, degp, b2):
    def body(r_ref, t_ref, dp_ref, b_ref, o_ref):
        s = r_ref[0] + r_ref[1] + t_ref[...]
        o_ref[...] = s * lax.rsqrt(_deg_col(dp_ref)) + b_ref[...]

    return pl.pallas_call(
        body,
        grid=(NP // _BR,),
        in_specs=[
            pl.BlockSpec((NSC, _BR, C), lambda i: (0, i, 0)),
            pl.BlockSpec((_BR, C), lambda i: (i, 0)),
            pl.BlockSpec((NSC, _BR, C), lambda i: (0, i, 0)),
            pl.BlockSpec((1, C), lambda i: (0, 0)),
        ],
        out_specs=pl.BlockSpec((_BR, C), lambda i: (i, 0)),
        out_shape=jax.ShapeDtypeStruct((NP, C), jnp.float32),
    )(rp, t, degp, b2)


def kernel(x, edge_index, W, b):
    E = edge_index.shape[1]
    burst = 10
    blk_edges = burst * CHUNK
    nblocks = -(-E // (NW * blk_edges))
    per_w = nblocks * blk_edges
    e_pad = NW * per_w

    ei = edge_index.astype(jnp.int32)
    pad = jnp.full((2, e_pad - E), N_NODES, jnp.int32)
    ei = jnp.concatenate([ei, pad], axis=1)
    src = ei[0].reshape(NW, nblocks, burst * CHUNK)
    dst = ei[1].reshape(NW, nblocks, burst * CHUNK)
    x_pad = jnp.pad(x, ((0, NP - x.shape[0]), (0, 0)))

    ones = jnp.ones((blk_edges, C), jnp.float32)
    degp = _sc_degree(nblocks, burst)(dst, ones)
    g = _tc_project(x_pad, W, degp)
    qp = _sc_propagate(nblocks, burst)(g, src, dst)
    t = _tc_combine(qp, g, degp)
    rp = _sc_propagate(nblocks, burst)(t, src, dst)
    out = _tc_final(rp, t, degp, b.reshape(1, C))
    return out[:N_NODES]
